# baseline (speedup 1.0000x reference)
import functools

import jax
import jax.numpy as jnp
from jax import lax
from jax.experimental import pallas as pl
from jax.experimental.pallas import tpu as pltpu
from jax.experimental.pallas import tpu_sc as plsc

_CODEBOOK = 1024
_DIM = 256
_BETA = 0.25
_T = 1024
_NTOK = 8192
_NW = 32
_BPW = _NTOK // _NW


def _argmin_body(nb, jt, z_ref, w_ref, idx_ref, loss_ref):
    b = pl.program_id(0)
    j = pl.program_id(1)
    zcm = z_ref[0]
    zf = zcm.T
    W = w_ref[...]

    a = jnp.sum(zf * zf, axis=1, keepdims=True)
    bb = jnp.sum(W * W, axis=1)
    m = jax.lax.dot_general(
        zf, W, dimension_numbers=(((1,), (1,)), ((), ())),
        preferred_element_type=jnp.float32)
    d = a + bb[None, :] - 2.0 * m

    dmin = jnp.min(d, axis=1, keepdims=True)
    iota = jax.lax.broadcasted_iota(jnp.int32, d.shape, 1)
    idx = jnp.min(jnp.where(d == dmin, iota, jnp.int32(_CODEBOOK)), axis=1)
    idx_ref[0, 0, 0] = idx

    partial = jnp.sum(dmin)
    first = jnp.logical_and(b == 0, j == 0)
    last = jnp.logical_and(b == nb - 1, j == jt - 1)
    prev = loss_ref[...]
    tot = jnp.where(first, partial, prev[0, 0] + partial)
    n_el = jnp.float32(_NTOK * _DIM)
    mean = tot / n_el
    loss_ref[...] = jnp.where(last, _BETA * mean + mean, tot).reshape(1, 1)


def _st_body(z_ref, zqt_ref, out_ref):
    zcm = z_ref[0]
    zqt = zqt_ref[0].T
    out_ref[0] = zcm + (zqt - zcm)


@functools.partial(
    pl.kernel,
    mesh=plsc.VectorSubcoreMesh(core_axis_name="c", subcore_axis_name="s"),
    out_type=jax.ShapeDtypeStruct((_NTOK, _DIM), jnp.float32),
    scratch_types=[
        pltpu.VMEM((_BPW,), jnp.int32),
        pltpu.VMEM((_BPW, _DIM), jnp.float32),
        pltpu.SemaphoreType.DMA,
    ],
)
def _sc_gather(w_hbm, idx_hbm, out_hbm, idx_v, rows_v, sem):
    wid = lax.axis_index("s") * 2 + lax.axis_index("c")
    base = wid * _BPW
    pltpu.sync_copy(idx_hbm.at[pl.ds(base, _BPW)], idx_v)
    pltpu.async_copy(w_hbm.at[idx_v], rows_v, sem).wait()
    pltpu.sync_copy(rows_v, out_hbm.at[pl.ds(base, _BPW)])


def kernel(z, W):
    B, C, H, Wd = z.shape
    hw = H * Wd
    zr = z.reshape(B, C, hw)
    nb, jt = B, hw // _T
    idx, loss = pl.pallas_call(
        functools.partial(_argmin_body, nb, jt),
        grid=(nb, jt),
        in_specs=[
            pl.BlockSpec((1, C, _T), lambda b, j: (b, 0, j)),
            pl.BlockSpec((_CODEBOOK, _DIM), lambda b, j: (0, 0)),
        ],
        out_specs=[
            pl.BlockSpec((1, 1, 1, _T), lambda b, j: (b, j, 0, 0)),
            pl.BlockSpec((1, 1), lambda b, j: (0, 0)),
        ],
        out_shape=[
            jax.ShapeDtypeStruct((nb, jt, 1, _T), jnp.int32),
            jax.ShapeDtypeStruct((1, 1), jnp.float32),
        ],
    )(zr, W)
    idx_flat = idx.reshape(B * hw)

    zq_tok = _sc_gather(W, idx_flat)

    zq = pl.pallas_call(
        _st_body,
        grid=(B,),
        in_specs=[
            pl.BlockSpec((1, C, hw), lambda b: (b, 0, 0)),
            pl.BlockSpec((1, hw, C), lambda b: (b, 0, 0)),
        ],
        out_specs=pl.BlockSpec((1, C, hw), lambda b: (b, 0, 0)),
        out_shape=jax.ShapeDtypeStruct((B, C, hw), jnp.float32),
    )(zr, zq_tok.reshape(B, hw, C))
    return (zq.reshape(B, C, H, Wd), loss.reshape(()), idx_flat)


# speedup vs baseline: 1.5241x; 1.5241x over previous
import functools

import jax
import jax.numpy as jnp
from jax.experimental import pallas as pl

_CODEBOOK = 1024
_DIM = 256
_BETA = 0.25
_T = 1024


def _vq_body(nb, jt, z_ref, w_ref, zq_ref, idx_ref, loss_ref):
    b = pl.program_id(0)
    j = pl.program_id(1)
    zcm = z_ref[0]
    zf = zcm.T
    W = w_ref[...]

    a = jnp.sum(zf * zf, axis=1, keepdims=True)
    bb = jnp.sum(W * W, axis=1)
    m = jax.lax.dot_general(
        zf, W, dimension_numbers=(((1,), (1,)), ((), ())),
        preferred_element_type=jnp.float32)
    d = a + bb[None, :] - 2.0 * m

    dmin = jnp.min(d, axis=1, keepdims=True)
    iota = jax.lax.broadcasted_iota(jnp.int32, d.shape, 1)
    idx = jnp.min(jnp.where(d == dmin, iota, jnp.int32(_CODEBOOK)), axis=1)

    onehot = (idx[:, None] == iota).astype(jnp.bfloat16)
    zq = jax.lax.dot_general(
        onehot, W.astype(jnp.bfloat16),
        dimension_numbers=(((1,), (0,)), ((), ())),
        preferred_element_type=jnp.float32)
    zqt = zq.T

    diff = zqt - zcm
    zq_ref[0] = zcm + diff
    idx_ref[0, 0, 0] = idx

    partial = jnp.sum(diff * diff)
    first = jnp.logical_and(b == 0, j == 0)
    last = jnp.logical_and(b == nb - 1, j == jt - 1)
    prev = loss_ref[...]
    tot = jnp.where(first, partial, prev[0, 0] + partial)
    n_el = jnp.float32(nb * jt * _T * _DIM)
    mean = tot / n_el
    loss_ref[...] = jnp.where(last, _BETA * mean + mean, tot).reshape(1, 1)


def kernel(z, W):
    B, C, H, Wd = z.shape
    hw = H * Wd
    zr = z.reshape(B, C, hw)
    nb, jt = B, hw // _T
    zq, idx, loss = pl.pallas_call(
        functools.partial(_vq_body, nb, jt),
        grid=(nb, jt),
        in_specs=[
            pl.BlockSpec((1, C, _T), lambda b, j: (b, 0, j)),
            pl.BlockSpec((_CODEBOOK, _DIM), lambda b, j: (0, 0)),
        ],
        out_specs=[
            pl.BlockSpec((1, C, _T), lambda b, j: (b, 0, j)),
            pl.BlockSpec((1, 1, 1, _T), lambda b, j: (b, j, 0, 0)),
            pl.BlockSpec((1, 1), lambda b, j: (0, 0)),
        ],
        out_shape=[
            jax.ShapeDtypeStruct((B, C, hw), jnp.float32),
            jax.ShapeDtypeStruct((nb, jt, 1, _T), jnp.int32),
            jax.ShapeDtypeStruct((1, 1), jnp.float32),
        ],
    )(zr, W)
    return (zq.reshape(B, C, H, Wd), loss.reshape(()), idx.reshape(B * hw))
